# baseline (speedup 1.0000x reference)
import math

import jax
import jax.numpy as jnp
from jax import lax
from jax.experimental import pallas as pl
from jax.experimental.pallas import tpu as pltpu

_NB = 16
_M = 8


def _adj_kernel(x_ref, q_ref, w1a_ref, w1b_ref, b1_ref, w2_ref, b2_ref, out_ref):
    cb, k, d = x_ref.shape
    x = x_ref[...].reshape(cb * k, d)
    h1 = jnp.dot(x, w1a_ref[...], preferred_element_type=jnp.float32)
    qp = jnp.dot(q_ref[...], w1b_ref[...], preferred_element_type=jnp.float32)
    h1 = h1.reshape(cb, k, h1.shape[-1]) + qp[:, None, :] + b1_ref[...]
    h1 = jnp.maximum(h1, 0.0).astype(jnp.bfloat16).reshape(cb * k, -1)
    h2 = jnp.dot(h1, w2_ref[...], preferred_element_type=jnp.float32) + b2_ref[...]
    h2 = jnp.maximum(h2, 0.0).astype(jnp.bfloat16)
    h2 = h2.reshape(cb, k, h2.shape[-1])
    for i in range(cb):
        hb = h2[i]
        out_ref[i] = lax.dot_general(hb, hb, (((1,), (1,)), ((), ())),
                                     preferred_element_type=jnp.float32)


def _adjacency(image_bf16, qenc_bf16, w1, b1, w2, b2, cb):
    b, k, d = image_bf16.shape
    h1 = w1.shape[1]
    w1a = w1[:d].astype(jnp.bfloat16)
    w1b = w1[d:].astype(jnp.bfloat16)
    return pl.pallas_call(
        _adj_kernel,
        out_shape=jax.ShapeDtypeStruct((b, k, k), jnp.float32),
        grid=(b // cb,),
        in_specs=[
            pl.BlockSpec((cb, k, d), lambda g: (g, 0, 0)),
            pl.BlockSpec((cb, qenc_bf16.shape[-1]), lambda g: (g, 0)),
            pl.BlockSpec(w1a.shape, lambda g: (0, 0)),
            pl.BlockSpec(w1b.shape, lambda g: (0, 0)),
            pl.BlockSpec((1, h1), lambda g: (0, 0)),
            pl.BlockSpec((h1, h1), lambda g: (0, 0)),
            pl.BlockSpec((1, h1), lambda g: (0, 0)),
        ],
        out_specs=pl.BlockSpec((cb, k, k), lambda g: (g, 0, 0)),
        compiler_params=pltpu.CompilerParams(dimension_semantics=("parallel",)),
    )(image_bf16, qenc_bf16, w1a, w1b, b1.reshape(1, h1), w2.astype(jnp.bfloat16),
      b2.reshape(1, h1))


def _tile_lanes(x, n):
    return jnp.concatenate([x] * n, axis=1)


def _topk_mask(adj_all, k):
    r = adj_all.shape[0]
    iota = lax.broadcasted_iota(jnp.int32, (r, k), 1)
    sel = jnp.zeros((r, k), jnp.float32)
    cur = adj_all
    for _ in range(_NB):
        jstar = jnp.argmax(cur, axis=1, keepdims=True)
        first = iota == jstar
        sel = jnp.where(first, 1.0, sel)
        cur = jnp.where(first, -jnp.inf, cur)
    return sel


def _gauss_block(rho_t, theta_t, g):
    dr = rho_t - g[0:1]
    first = jnp.abs(theta_t - g[1:2])
    ang = jnp.minimum(first, 2.0 * math.pi - first)
    return jnp.exp2(dr * dr * g[2:3] + ang * ang * g[3:4])


def _wsum(w_all, k):
    s = w_all[:, 0:k]
    for m in range(1, _M):
        s = s + w_all[:, m * k:(m + 1) * k]
    return s


def _gc_head_kernel(adj_ref, feats_ref, cc_ref, qenc_ref,
                    g1_ref, g2_ref, wc1_ref, wc2_ref, bm1_ref, bm2_ref,
                    w1_ref, b1_ref, w2_ref, b2_ref, out_ref):
    cb, k, _ = adj_ref.shape
    d_in = feats_ref.shape[-1]
    adj_all = adj_ref[...].reshape(cb * k, k)

    sel_all = _topk_mask(adj_all, k)
    rowmax = jnp.max(adj_all, axis=1, keepdims=True)
    e = sel_all * jnp.exp(adj_all - rowmax)
    a1_all = e / jnp.sum(e, axis=1, keepdims=True)

    rhos, thetas = [], []
    for i in range(cb):
        cc = cc_ref[i]
        cxm = jnp.broadcast_to(cc[0:1, :], (k, k))
        cym = jnp.broadcast_to(cc[1:2, :], (k, k))
        dx = jnp.transpose(cxm) - cxm
        dy = jnp.transpose(cym) - cym
        rhos.append(jnp.sqrt(dx * dx + dy * dy))
        thetas.append(jnp.arctan2(dx, dy))
    rho_t = _tile_lanes(jnp.concatenate(rhos, axis=0), _M)
    theta_t = _tile_lanes(jnp.concatenate(thetas, axis=0), _M)

    w1_all = _gauss_block(rho_t, theta_t, g1_ref[...])
    w2_all = _gauss_block(rho_t, theta_t, g2_ref[...])
    b1_stack = w1_all * _tile_lanes(a1_all / (_wsum(w1_all, k) + 1e-20), _M)
    b2_stack = w2_all * _tile_lanes(sel_all / (_wsum(w2_all, k) + 1e-20), _M)

    proj1 = jnp.dot(feats_ref[...].reshape(cb * k, d_in), wc1_ref[...],
                    preferred_element_type=jnp.float32)
    bm1 = bm1_ref[...]
    bm2 = bm2_ref[...]
    hg1s = []
    for i in range(cb):
        p = jnp.concatenate([proj1[i * k:(i + 1) * k]] * _M, axis=0) * bm1
        o = jnp.dot(b1_stack[i * k:(i + 1) * k], p,
                    preferred_element_type=jnp.float32)
        hg1s.append(jnp.maximum(o, 0.0))
    hg1_all = jnp.concatenate(hg1s, axis=0)

    proj2 = jnp.dot(hg1_all, wc2_ref[...],
                    preferred_element_type=jnp.float32)
    gate_rows = []
    for i in range(cb):
        p = jnp.concatenate([proj2[i * k:(i + 1) * k]] * _M, axis=0) * bm2
        o = jnp.dot(b2_stack[i * k:(i + 1) * k], p,
                    preferred_element_type=jnp.float32)
        gate_rows.append(jnp.max(jnp.maximum(o, 0.0), axis=0, keepdims=True))

    q = jnp.maximum(qenc_ref[...], 0.0)
    h = q * jnp.concatenate(gate_rows, axis=0)
    h1 = jnp.dot(h, w1_ref[...], preferred_element_type=jnp.float32) + b1_ref[...]
    h1 = jnp.maximum(h1, 0.0)
    out_ref[...] = jnp.dot(h1, w2_ref[...],
                           preferred_element_type=jnp.float32) + b2_ref[...]


def _gauss_rows(mean_rho, mean_theta, prec_rho, prec_theta, k):
    c = -0.5 * math.log2(math.e)
    cr = c / (1e-14 + prec_rho ** 2)
    ct = c / (1e-14 + prec_theta ** 2)
    return jnp.stack([jnp.repeat(mean_rho, k), jnp.repeat(mean_theta, k),
                      jnp.repeat(cr, k), jnp.repeat(ct, k)], axis=0)


def _block_mask(k, dout):
    opk = dout // _M
    eye = jnp.eye(_M, dtype=jnp.float32)
    return jnp.repeat(jnp.repeat(eye, k, axis=0), opk, axis=1)


def _gc_head(adjacency, image, cc, qenc, gc1, gc2, w1, b1, w2, b2, cb):
    b, k, _ = adjacency.shape
    d_in = image.shape[-1]
    wc1 = jnp.transpose(gc1["conv_w"], (1, 0, 2)).reshape(d_in, -1)
    d1 = wc1.shape[1]
    wc2 = jnp.transpose(gc2["conv_w"], (1, 0, 2)).reshape(d1, -1)
    h = wc2.shape[1]
    o = w1.shape[0]
    g1 = _gauss_rows(gc1["mean_rho"], gc1["mean_theta"], gc1["prec_rho"],
                     gc1["prec_theta"], k)
    g2 = _gauss_rows(gc2["mean_rho"], gc2["mean_theta"], gc2["prec_rho"],
                     gc2["prec_theta"], k)
    bm1 = _block_mask(k, d1)
    bm2 = _block_mask(k, h)
    return pl.pallas_call(
        _gc_head_kernel,
        out_shape=jax.ShapeDtypeStruct((b, o), jnp.float32),
        grid=(b // cb,),
        in_specs=[
            pl.BlockSpec((cb, k, k), lambda g: (g, 0, 0)),
            pl.BlockSpec((cb, k, d_in), lambda g: (g, 0, 0)),
            pl.BlockSpec((cb, 2, k), lambda g: (g, 0, 0)),
            pl.BlockSpec((cb, h), lambda g: (g, 0)),
            pl.BlockSpec((4, _M * k), lambda g: (0, 0)),
            pl.BlockSpec((4, _M * k), lambda g: (0, 0)),
            pl.BlockSpec((d_in, d1), lambda g: (0, 0)),
            pl.BlockSpec((d1, h), lambda g: (0, 0)),
            pl.BlockSpec((_M * k, d1), lambda g: (0, 0)),
            pl.BlockSpec((_M * k, h), lambda g: (0, 0)),
            pl.BlockSpec((h, o), lambda g: (0, 0)),
            pl.BlockSpec((1, o), lambda g: (0, 0)),
            pl.BlockSpec((o, o), lambda g: (0, 0)),
            pl.BlockSpec((1, o), lambda g: (0, 0)),
        ],
        out_specs=pl.BlockSpec((cb, o), lambda g: (g, 0)),
        compiler_params=pltpu.CompilerParams(dimension_semantics=("parallel",)),
    )(adjacency, image, cc, qenc, g1, g2, wc1, wc2, bm1, bm2,
      jnp.transpose(w1), b1.reshape(1, o), jnp.transpose(w2), b2.reshape(1, o))


def _gru_kernel(emb_ref, qlen_ref, wih_ref, whh_ref, bih_ref, bhh_ref, out_ref):
    gb, t, e = emb_ref.shape
    hdim = whh_ref.shape[0]
    qlen = qlen_ref[...]
    hs = jnp.zeros((gb, hdim), jnp.float32)
    for tt in range(t):
        xt = emb_ref[:, tt, :]
        gi = jnp.dot(xt, wih_ref[...], preferred_element_type=jnp.float32) + bih_ref[...]
        gh = jnp.dot(hs, whh_ref[...], preferred_element_type=jnp.float32) + bhh_ref[...]
        i_r, i_z, i_n = gi[:, :hdim], gi[:, hdim:2 * hdim], gi[:, 2 * hdim:]
        h_r, h_z, h_n = gh[:, :hdim], gh[:, hdim:2 * hdim], gh[:, 2 * hdim:]
        r = jax.nn.sigmoid(i_r + h_r)
        z = jax.nn.sigmoid(i_z + h_z)
        n = jnp.tanh(i_n + r * h_n)
        h_new = (1.0 - z) * n + z * hs
        hs = jnp.where(tt < qlen, h_new, hs)
    out_ref[...] = hs


def _gru_final_hidden(emb, qlen, w_ih, w_hh, b_ih, b_hh):
    b, t, e = emb.shape
    hdim = w_hh.shape[1]
    gb = 1024 if b % 1024 == 0 else b
    return pl.pallas_call(
        _gru_kernel,
        out_shape=jax.ShapeDtypeStruct((b, hdim), jnp.float32),
        grid=(b // gb,),
        in_specs=[
            pl.BlockSpec((gb, t, e), lambda g: (g, 0, 0)),
            pl.BlockSpec((gb, 1), lambda g: (g, 0)),
            pl.BlockSpec((e, 3 * hdim), lambda g: (0, 0)),
            pl.BlockSpec((hdim, 3 * hdim), lambda g: (0, 0)),
            pl.BlockSpec((1, 3 * hdim), lambda g: (0, 0)),
            pl.BlockSpec((1, 3 * hdim), lambda g: (0, 0)),
        ],
        out_specs=pl.BlockSpec((gb, hdim), lambda g: (g, 0)),
        compiler_params=pltpu.CompilerParams(dimension_semantics=("parallel",)),
    )(emb, qlen.reshape(b, 1), jnp.transpose(w_ih), jnp.transpose(w_hh),
      b_ih.reshape(1, -1), b_hh.reshape(1, -1))


def kernel(wembed, gru_w_ih, gru_w_hh, gru_b_ih, gru_b_hh,
           adj_w1, adj_b1, adj_w2, adj_b2,
           gc1_conv_w, gc1_mean_rho, gc1_mean_theta, gc1_prec_rho, gc1_prec_theta,
           gc2_conv_w, gc2_mean_rho, gc2_mean_theta, gc2_prec_rho, gc2_prec_theta,
           out1_w, out1_b, out2_w, out2_b,
           question, image, qlen):
    b, k, _ = image.shape
    cb = 16 if b % 16 == 0 else 1

    emb = wembed[question]
    qenc = _gru_final_hidden(emb, qlen, gru_w_ih, gru_w_hh, gru_b_ih, gru_b_hh)

    adjacency = _adjacency(image.astype(jnp.bfloat16), qenc.astype(jnp.bfloat16),
                           adj_w1, adj_b1, adj_w2, adj_b2, cb)

    bb = image[:, :, -4:]
    centre = bb[:, :, :2] + 0.5 * (bb[:, :, 2:] - bb[:, :, :2])
    cc = jnp.transpose(centre, (0, 2, 1))

    gc1 = {"conv_w": gc1_conv_w, "mean_rho": gc1_mean_rho,
           "mean_theta": gc1_mean_theta, "prec_rho": gc1_prec_rho,
           "prec_theta": gc1_prec_theta}
    gc2 = {"conv_w": gc2_conv_w, "mean_rho": gc2_mean_rho,
           "mean_theta": gc2_mean_theta, "prec_rho": gc2_prec_rho,
           "prec_theta": gc2_prec_theta}
    logits = _gc_head(adjacency, image, cc, qenc, gc1, gc2,
                      out1_w, out1_b, out2_w, out2_b, cb)
    return logits, adjacency


# speedup vs baseline: 1.0349x; 1.0349x over previous
import math

import jax
import jax.numpy as jnp
from jax import lax
from jax.experimental import pallas as pl
from jax.experimental.pallas import tpu as pltpu

_NB = 16
_M = 8


def _adj_kernel(x_ref, q_ref, w1a_ref, w1b_ref, b1_ref, w2_ref, b2_ref, out_ref):
    cb, k, d = x_ref.shape
    x = x_ref[...].reshape(cb * k, d)
    h1 = jnp.dot(x, w1a_ref[...], preferred_element_type=jnp.float32)
    qp = jnp.dot(q_ref[...], w1b_ref[...], preferred_element_type=jnp.float32)
    h1 = h1.reshape(cb, k, h1.shape[-1]) + qp[:, None, :] + b1_ref[...]
    h1 = jnp.maximum(h1, 0.0).astype(jnp.bfloat16).reshape(cb * k, -1)
    h2 = jnp.dot(h1, w2_ref[...], preferred_element_type=jnp.float32) + b2_ref[...]
    h2 = jnp.maximum(h2, 0.0).astype(jnp.bfloat16)
    h2 = h2.reshape(cb, k, h2.shape[-1])
    for i in range(cb):
        hb = h2[i]
        out_ref[i] = lax.dot_general(hb, hb, (((1,), (1,)), ((), ())),
                                     preferred_element_type=jnp.float32)


def _adjacency(image_bf16, qenc_bf16, w1, b1, w2, b2, cb):
    b, k, d = image_bf16.shape
    h1 = w1.shape[1]
    w1a = w1[:d].astype(jnp.bfloat16)
    w1b = w1[d:].astype(jnp.bfloat16)
    return pl.pallas_call(
        _adj_kernel,
        out_shape=jax.ShapeDtypeStruct((b, k, k), jnp.float32),
        grid=(b // cb,),
        in_specs=[
            pl.BlockSpec((cb, k, d), lambda g: (g, 0, 0)),
            pl.BlockSpec((cb, qenc_bf16.shape[-1]), lambda g: (g, 0)),
            pl.BlockSpec(w1a.shape, lambda g: (0, 0)),
            pl.BlockSpec(w1b.shape, lambda g: (0, 0)),
            pl.BlockSpec((1, h1), lambda g: (0, 0)),
            pl.BlockSpec((h1, h1), lambda g: (0, 0)),
            pl.BlockSpec((1, h1), lambda g: (0, 0)),
        ],
        out_specs=pl.BlockSpec((cb, k, k), lambda g: (g, 0, 0)),
        compiler_params=pltpu.CompilerParams(dimension_semantics=("parallel",)),
    )(image_bf16, qenc_bf16, w1a, w1b, b1.reshape(1, h1), w2.astype(jnp.bfloat16),
      b2.reshape(1, h1))


def _tile_lanes(x, n):
    return jnp.concatenate([x] * n, axis=1)


def _topk_mask(adj_all, k):
    r = adj_all.shape[0]
    iota = lax.broadcasted_iota(jnp.int32, (r, k), 1)
    sel = jnp.zeros((r, k), jnp.float32)
    cur = adj_all
    for _ in range(_NB):
        jstar = jnp.argmax(cur, axis=1, keepdims=True)
        first = iota == jstar
        sel = jnp.where(first, 1.0, sel)
        cur = jnp.where(first, -jnp.inf, cur)
    return sel


def _gauss_block(rho_t, theta_t, g):
    dr = rho_t - g[0:1]
    first = jnp.abs(theta_t - g[1:2])
    ang = jnp.minimum(first, 2.0 * math.pi - first)
    return jnp.exp2(dr * dr * g[2:3] + ang * ang * g[3:4])


def _wsum(w_all, k):
    s = w_all[:, 0:k]
    for m in range(1, _M):
        s = s + w_all[:, m * k:(m + 1) * k]
    return s


def _gc_head_kernel(adj_ref, feats_ref, cc_ref, qenc_ref,
                    g1_ref, g2_ref, wc1_ref, wc2_ref, bm1_ref, bm2_ref,
                    w1_ref, b1_ref, w2_ref, b2_ref, out_ref):
    cb, k, _ = adj_ref.shape
    d_in = feats_ref.shape[-1]
    adj_all = adj_ref[...].reshape(cb * k, k)

    sel_all = _topk_mask(adj_all, k)
    rowmax = jnp.max(adj_all, axis=1, keepdims=True)
    e = sel_all * jnp.exp(adj_all - rowmax)
    a1_all = e / jnp.sum(e, axis=1, keepdims=True)

    rhos, thetas = [], []
    for i in range(cb):
        cc = cc_ref[i]
        cxm = jnp.broadcast_to(cc[0:1, :], (k, k))
        cym = jnp.broadcast_to(cc[1:2, :], (k, k))
        dx = jnp.transpose(cxm) - cxm
        dy = jnp.transpose(cym) - cym
        rhos.append(jnp.sqrt(dx * dx + dy * dy))
        thetas.append(jnp.arctan2(dx, dy))
    rho_t = _tile_lanes(jnp.concatenate(rhos, axis=0), _M)
    theta_t = _tile_lanes(jnp.concatenate(thetas, axis=0), _M)

    w1_all = _gauss_block(rho_t, theta_t, g1_ref[...])
    w2_all = _gauss_block(rho_t, theta_t, g2_ref[...])
    b1_stack = w1_all * _tile_lanes(a1_all / (_wsum(w1_all, k) + 1e-20), _M)
    b2_stack = w2_all * _tile_lanes(sel_all / (_wsum(w2_all, k) + 1e-20), _M)

    proj1 = jnp.dot(feats_ref[...].reshape(cb * k, d_in), wc1_ref[...],
                    preferred_element_type=jnp.float32)
    bm1 = bm1_ref[...]
    bm2 = bm2_ref[...]
    hg1s = []
    for i in range(cb):
        p = jnp.concatenate([proj1[i * k:(i + 1) * k]] * _M, axis=0) * bm1
        o = jnp.dot(b1_stack[i * k:(i + 1) * k], p,
                    preferred_element_type=jnp.float32)
        hg1s.append(jnp.maximum(o, 0.0))
    hg1_all = jnp.concatenate(hg1s, axis=0)

    proj2 = jnp.dot(hg1_all, wc2_ref[...],
                    preferred_element_type=jnp.float32)
    gate_rows = []
    for i in range(cb):
        p = jnp.concatenate([proj2[i * k:(i + 1) * k]] * _M, axis=0) * bm2
        o = jnp.dot(b2_stack[i * k:(i + 1) * k], p,
                    preferred_element_type=jnp.float32)
        gate_rows.append(jnp.max(jnp.maximum(o, 0.0), axis=0, keepdims=True))

    q = jnp.maximum(qenc_ref[...], 0.0)
    h = q * jnp.concatenate(gate_rows, axis=0)
    h1 = jnp.dot(h, w1_ref[...], preferred_element_type=jnp.float32) + b1_ref[...]
    h1 = jnp.maximum(h1, 0.0)
    out_ref[...] = jnp.dot(h1, w2_ref[...],
                           preferred_element_type=jnp.float32) + b2_ref[...]


def _gauss_rows(mean_rho, mean_theta, prec_rho, prec_theta, k):
    c = -0.5 * math.log2(math.e)
    cr = c / (1e-14 + prec_rho ** 2)
    ct = c / (1e-14 + prec_theta ** 2)
    return jnp.stack([jnp.repeat(mean_rho, k), jnp.repeat(mean_theta, k),
                      jnp.repeat(cr, k), jnp.repeat(ct, k)], axis=0)


def _block_mask(k, dout):
    opk = dout // _M
    eye = jnp.eye(_M, dtype=jnp.float32)
    return jnp.repeat(jnp.repeat(eye, k, axis=0), opk, axis=1)


def _gc_head(adjacency, image, cc, qenc, gc1, gc2, w1, b1, w2, b2, cb):
    b, k, _ = adjacency.shape
    d_in = image.shape[-1]
    wc1 = jnp.transpose(gc1["conv_w"], (1, 0, 2)).reshape(d_in, -1)
    d1 = wc1.shape[1]
    wc2 = jnp.transpose(gc2["conv_w"], (1, 0, 2)).reshape(d1, -1)
    h = wc2.shape[1]
    o = w1.shape[0]
    g1 = _gauss_rows(gc1["mean_rho"], gc1["mean_theta"], gc1["prec_rho"],
                     gc1["prec_theta"], k)
    g2 = _gauss_rows(gc2["mean_rho"], gc2["mean_theta"], gc2["prec_rho"],
                     gc2["prec_theta"], k)
    bm1 = _block_mask(k, d1)
    bm2 = _block_mask(k, h)
    return pl.pallas_call(
        _gc_head_kernel,
        out_shape=jax.ShapeDtypeStruct((b, o), jnp.float32),
        grid=(b // cb,),
        in_specs=[
            pl.BlockSpec((cb, k, k), lambda g: (g, 0, 0)),
            pl.BlockSpec((cb, k, d_in), lambda g: (g, 0, 0)),
            pl.BlockSpec((cb, 2, k), lambda g: (g, 0, 0)),
            pl.BlockSpec((cb, h), lambda g: (g, 0)),
            pl.BlockSpec((4, _M * k), lambda g: (0, 0)),
            pl.BlockSpec((4, _M * k), lambda g: (0, 0)),
            pl.BlockSpec((d_in, d1), lambda g: (0, 0)),
            pl.BlockSpec((d1, h), lambda g: (0, 0)),
            pl.BlockSpec((_M * k, d1), lambda g: (0, 0)),
            pl.BlockSpec((_M * k, h), lambda g: (0, 0)),
            pl.BlockSpec((h, o), lambda g: (0, 0)),
            pl.BlockSpec((1, o), lambda g: (0, 0)),
            pl.BlockSpec((o, o), lambda g: (0, 0)),
            pl.BlockSpec((1, o), lambda g: (0, 0)),
        ],
        out_specs=pl.BlockSpec((cb, o), lambda g: (g, 0)),
        compiler_params=pltpu.CompilerParams(dimension_semantics=("parallel",)),
    )(adjacency, image, cc, qenc, g1, g2, wc1, wc2, bm1, bm2,
      jnp.transpose(w1), b1.reshape(1, o), jnp.transpose(w2), b2.reshape(1, o))


def _gru_kernel(emb_ref, qlen_ref, wih_ref, whh_ref, bih_ref, bhh_ref, out_ref):
    gb, t, e = emb_ref.shape
    hdim = whh_ref.shape[0]
    qlen = qlen_ref[...]
    hs = jnp.zeros((gb, hdim), jnp.float32)
    for tt in range(t):
        xt = emb_ref[:, tt, :]
        gi = jnp.dot(xt, wih_ref[...], preferred_element_type=jnp.float32) + bih_ref[...]
        gh = jnp.dot(hs, whh_ref[...], preferred_element_type=jnp.float32) + bhh_ref[...]
        i_r, i_z, i_n = gi[:, :hdim], gi[:, hdim:2 * hdim], gi[:, 2 * hdim:]
        h_r, h_z, h_n = gh[:, :hdim], gh[:, hdim:2 * hdim], gh[:, 2 * hdim:]
        r = jax.nn.sigmoid(i_r + h_r)
        z = jax.nn.sigmoid(i_z + h_z)
        n = jnp.tanh(i_n + r * h_n)
        h_new = (1.0 - z) * n + z * hs
        hs = jnp.where(tt < qlen, h_new, hs)
    out_ref[...] = hs


def _gru_final_hidden(emb, qlen, w_ih, w_hh, b_ih, b_hh):
    b, t, e = emb.shape
    hdim = w_hh.shape[1]
    gb = 1024 if b % 1024 == 0 else b
    return pl.pallas_call(
        _gru_kernel,
        out_shape=jax.ShapeDtypeStruct((b, hdim), jnp.float32),
        grid=(b // gb,),
        in_specs=[
            pl.BlockSpec((gb, t, e), lambda g: (g, 0, 0)),
            pl.BlockSpec((gb, 1), lambda g: (g, 0)),
            pl.BlockSpec((e, 3 * hdim), lambda g: (0, 0)),
            pl.BlockSpec((hdim, 3 * hdim), lambda g: (0, 0)),
            pl.BlockSpec((1, 3 * hdim), lambda g: (0, 0)),
            pl.BlockSpec((1, 3 * hdim), lambda g: (0, 0)),
        ],
        out_specs=pl.BlockSpec((gb, hdim), lambda g: (g, 0)),
        compiler_params=pltpu.CompilerParams(dimension_semantics=("parallel",)),
    )(emb, qlen.reshape(b, 1), jnp.transpose(w_ih), jnp.transpose(w_hh),
      b_ih.reshape(1, -1), b_hh.reshape(1, -1))


def kernel(wembed, gru_w_ih, gru_w_hh, gru_b_ih, gru_b_hh,
           adj_w1, adj_b1, adj_w2, adj_b2,
           gc1_conv_w, gc1_mean_rho, gc1_mean_theta, gc1_prec_rho, gc1_prec_theta,
           gc2_conv_w, gc2_mean_rho, gc2_mean_theta, gc2_prec_rho, gc2_prec_theta,
           out1_w, out1_b, out2_w, out2_b,
           question, image, qlen):
    b, k, _ = image.shape
    cb = 64 if b % 64 == 0 else 1

    emb = wembed[question]
    qenc = _gru_final_hidden(emb, qlen, gru_w_ih, gru_w_hh, gru_b_ih, gru_b_hh)

    adjacency = _adjacency(image.astype(jnp.bfloat16), qenc.astype(jnp.bfloat16),
                           adj_w1, adj_b1, adj_w2, adj_b2, cb)

    bb = image[:, :, -4:]
    centre = bb[:, :, :2] + 0.5 * (bb[:, :, 2:] - bb[:, :, :2])
    cc = jnp.transpose(centre, (0, 2, 1))

    gc1 = {"conv_w": gc1_conv_w, "mean_rho": gc1_mean_rho,
           "mean_theta": gc1_mean_theta, "prec_rho": gc1_prec_rho,
           "prec_theta": gc1_prec_theta}
    gc2 = {"conv_w": gc2_conv_w, "mean_rho": gc2_mean_rho,
           "mean_theta": gc2_mean_theta, "prec_rho": gc2_prec_rho,
           "prec_theta": gc2_prec_theta}
    logits = _gc_head(adjacency, image, cc, qenc, gc1, gc2,
                      out1_w, out1_b, out2_w, out2_b, cb)
    return logits, adjacency


# speedup vs baseline: 1.0892x; 1.0524x over previous
import math

import jax
import jax.numpy as jnp
from jax import lax
from jax.experimental import pallas as pl
from jax.experimental.pallas import tpu as pltpu

_NB = 16
_M = 8


def _adj_kernel(x_ref, q_ref, w1a_ref, w1b_ref, b1_ref, w2_ref, b2_ref, out_ref):
    cb, k, d = x_ref.shape
    x = x_ref[...].astype(jnp.bfloat16).reshape(cb * k, d)
    h1 = jnp.dot(x, w1a_ref[...], preferred_element_type=jnp.float32)
    qp = jnp.dot(q_ref[...].astype(jnp.bfloat16), w1b_ref[...],
                 preferred_element_type=jnp.float32)
    h1 = h1.reshape(cb, k, h1.shape[-1]) + qp[:, None, :] + b1_ref[...]
    h1 = jnp.maximum(h1, 0.0).astype(jnp.bfloat16).reshape(cb * k, -1)
    h2 = jnp.dot(h1, w2_ref[...], preferred_element_type=jnp.float32) + b2_ref[...]
    h2 = jnp.maximum(h2, 0.0).astype(jnp.bfloat16)
    h2 = h2.reshape(cb, k, h2.shape[-1])
    for i in range(cb):
        hb = h2[i]
        out_ref[i] = lax.dot_general(hb, hb, (((1,), (1,)), ((), ())),
                                     preferred_element_type=jnp.float32)


def _adjacency(image, qenc, w1, b1, w2, b2, cb):
    b, k, d = image.shape
    h1 = w1.shape[1]
    w1a = w1[:d].astype(jnp.bfloat16)
    w1b = w1[d:].astype(jnp.bfloat16)
    return pl.pallas_call(
        _adj_kernel,
        out_shape=jax.ShapeDtypeStruct((b, k, k), jnp.float32),
        grid=(b // cb,),
        in_specs=[
            pl.BlockSpec((cb, k, d), lambda g: (g, 0, 0)),
            pl.BlockSpec((cb, qenc.shape[-1]), lambda g: (g, 0)),
            pl.BlockSpec(w1a.shape, lambda g: (0, 0)),
            pl.BlockSpec(w1b.shape, lambda g: (0, 0)),
            pl.BlockSpec((1, h1), lambda g: (0, 0)),
            pl.BlockSpec((h1, h1), lambda g: (0, 0)),
            pl.BlockSpec((1, h1), lambda g: (0, 0)),
        ],
        out_specs=pl.BlockSpec((cb, k, k), lambda g: (g, 0, 0)),
        compiler_params=pltpu.CompilerParams(dimension_semantics=("parallel",)),
    )(image, qenc, w1a, w1b, b1.reshape(1, h1), w2.astype(jnp.bfloat16),
      b2.reshape(1, h1))


def _tile_lanes(x, n):
    return jnp.concatenate([x] * n, axis=1)


def _topk_mask(adj_all, k):
    r = adj_all.shape[0]
    iota = lax.broadcasted_iota(jnp.int32, (r, k), 1)
    sel = jnp.zeros((r, k), jnp.float32)
    cur = adj_all
    for _ in range(_NB):
        jstar = jnp.argmax(cur, axis=1, keepdims=True)
        first = iota == jstar
        sel = jnp.where(first, 1.0, sel)
        cur = jnp.where(first, -jnp.inf, cur)
    return sel


def _gauss_block(rho_t, theta_t, g):
    dr = rho_t - g[0:1]
    first = jnp.abs(theta_t - g[1:2])
    ang = jnp.minimum(first, 2.0 * math.pi - first)
    return jnp.exp2(dr * dr * g[2:3] + ang * ang * g[3:4])


def _wsum(w_all, k):
    s = w_all[:, 0:k]
    for m in range(1, _M):
        s = s + w_all[:, m * k:(m + 1) * k]
    return s


def _gc_head_kernel(adj_ref, feats_ref, cc_ref, qenc_ref,
                    g1_ref, g2_ref, wc1_ref, wc2_ref, bm1_ref, bm2_ref,
                    w1_ref, b1_ref, w2_ref, b2_ref, out_ref):
    cb, k, _ = adj_ref.shape
    d_in = feats_ref.shape[-1]
    adj_all = adj_ref[...].reshape(cb * k, k)

    sel_all = _topk_mask(adj_all, k)
    rowmax = jnp.max(adj_all, axis=1, keepdims=True)
    e = sel_all * jnp.exp(adj_all - rowmax)
    a1_all = e / jnp.sum(e, axis=1, keepdims=True)

    rhos, thetas = [], []
    for i in range(cb):
        cc = cc_ref[i]
        cxm = jnp.broadcast_to(cc[0:1, :], (k, k))
        cym = jnp.broadcast_to(cc[1:2, :], (k, k))
        dx = jnp.transpose(cxm) - cxm
        dy = jnp.transpose(cym) - cym
        rhos.append(jnp.sqrt(dx * dx + dy * dy))
        thetas.append(jnp.arctan2(dx, dy))
    rho_t = _tile_lanes(jnp.concatenate(rhos, axis=0), _M)
    theta_t = _tile_lanes(jnp.concatenate(thetas, axis=0), _M)

    w1_all = _gauss_block(rho_t, theta_t, g1_ref[...])
    w2_all = _gauss_block(rho_t, theta_t, g2_ref[...])
    b1_stack = w1_all * _tile_lanes(a1_all / (_wsum(w1_all, k) + 1e-20), _M)
    b2_stack = w2_all * _tile_lanes(sel_all / (_wsum(w2_all, k) + 1e-20), _M)

    proj1 = jnp.dot(feats_ref[...].reshape(cb * k, d_in), wc1_ref[...],
                    preferred_element_type=jnp.float32)
    bm1 = bm1_ref[...]
    bm2 = bm2_ref[...]
    hg1s = []
    for i in range(cb):
        p = jnp.concatenate([proj1[i * k:(i + 1) * k]] * _M, axis=0) * bm1
        o = jnp.dot(b1_stack[i * k:(i + 1) * k], p,
                    preferred_element_type=jnp.float32)
        hg1s.append(jnp.maximum(o, 0.0))
    hg1_all = jnp.concatenate(hg1s, axis=0)

    proj2 = jnp.dot(hg1_all, wc2_ref[...],
                    preferred_element_type=jnp.float32)
    gate_rows = []
    for i in range(cb):
        p = jnp.concatenate([proj2[i * k:(i + 1) * k]] * _M, axis=0) * bm2
        o = jnp.dot(b2_stack[i * k:(i + 1) * k], p,
                    preferred_element_type=jnp.float32)
        gate_rows.append(jnp.max(jnp.maximum(o, 0.0), axis=0, keepdims=True))

    q = jnp.maximum(qenc_ref[...], 0.0)
    h = q * jnp.concatenate(gate_rows, axis=0)
    h1 = jnp.dot(h, w1_ref[...], preferred_element_type=jnp.float32) + b1_ref[...]
    h1 = jnp.maximum(h1, 0.0)
    out_ref[...] = jnp.dot(h1, w2_ref[...],
                           preferred_element_type=jnp.float32) + b2_ref[...]


def _gauss_rows(mean_rho, mean_theta, prec_rho, prec_theta, k):
    c = -0.5 * math.log2(math.e)
    cr = c / (1e-14 + prec_rho ** 2)
    ct = c / (1e-14 + prec_theta ** 2)
    return jnp.stack([jnp.repeat(mean_rho, k), jnp.repeat(mean_theta, k),
                      jnp.repeat(cr, k), jnp.repeat(ct, k)], axis=0)


def _block_mask(k, dout):
    opk = dout // _M
    eye = jnp.eye(_M, dtype=jnp.float32)
    return jnp.repeat(jnp.repeat(eye, k, axis=0), opk, axis=1)


def _gc_head(adjacency, image, cc, qenc, gc1, gc2, w1, b1, w2, b2, cb):
    b, k, _ = adjacency.shape
    d_in = image.shape[-1]
    wc1 = jnp.transpose(gc1["conv_w"], (1, 0, 2)).reshape(d_in, -1)
    d1 = wc1.shape[1]
    wc2 = jnp.transpose(gc2["conv_w"], (1, 0, 2)).reshape(d1, -1)
    h = wc2.shape[1]
    o = w1.shape[0]
    g1 = _gauss_rows(gc1["mean_rho"], gc1["mean_theta"], gc1["prec_rho"],
                     gc1["prec_theta"], k)
    g2 = _gauss_rows(gc2["mean_rho"], gc2["mean_theta"], gc2["prec_rho"],
                     gc2["prec_theta"], k)
    bm1 = _block_mask(k, d1)
    bm2 = _block_mask(k, h)
    return pl.pallas_call(
        _gc_head_kernel,
        out_shape=jax.ShapeDtypeStruct((b, o), jnp.float32),
        grid=(b // cb,),
        in_specs=[
            pl.BlockSpec((cb, k, k), lambda g: (g, 0, 0)),
            pl.BlockSpec((cb, k, d_in), lambda g: (g, 0, 0)),
            pl.BlockSpec((cb, 2, k), lambda g: (g, 0, 0)),
            pl.BlockSpec((cb, h), lambda g: (g, 0)),
            pl.BlockSpec((4, _M * k), lambda g: (0, 0)),
            pl.BlockSpec((4, _M * k), lambda g: (0, 0)),
            pl.BlockSpec((d_in, d1), lambda g: (0, 0)),
            pl.BlockSpec((d1, h), lambda g: (0, 0)),
            pl.BlockSpec((_M * k, d1), lambda g: (0, 0)),
            pl.BlockSpec((_M * k, h), lambda g: (0, 0)),
            pl.BlockSpec((h, o), lambda g: (0, 0)),
            pl.BlockSpec((1, o), lambda g: (0, 0)),
            pl.BlockSpec((o, o), lambda g: (0, 0)),
            pl.BlockSpec((1, o), lambda g: (0, 0)),
        ],
        out_specs=pl.BlockSpec((cb, o), lambda g: (g, 0)),
        compiler_params=pltpu.CompilerParams(dimension_semantics=("parallel",)),
    )(adjacency, image, cc, qenc, g1, g2, wc1, wc2, bm1, bm2,
      jnp.transpose(w1), b1.reshape(1, o), jnp.transpose(w2), b2.reshape(1, o))


def _gru_kernel(emb_ref, qlen_ref, wih_ref, whh_ref, bih_ref, bhh_ref, out_ref):
    gb, t, e = emb_ref.shape
    hdim = whh_ref.shape[0]
    qlen = qlen_ref[...]
    hs = jnp.zeros((gb, hdim), jnp.float32)
    for tt in range(t):
        xt = emb_ref[:, tt, :]
        gi = jnp.dot(xt, wih_ref[...], preferred_element_type=jnp.float32) + bih_ref[...]
        gh = jnp.dot(hs, whh_ref[...], preferred_element_type=jnp.float32) + bhh_ref[...]
        i_r, i_z, i_n = gi[:, :hdim], gi[:, hdim:2 * hdim], gi[:, 2 * hdim:]
        h_r, h_z, h_n = gh[:, :hdim], gh[:, hdim:2 * hdim], gh[:, 2 * hdim:]
        r = jax.nn.sigmoid(i_r + h_r)
        z = jax.nn.sigmoid(i_z + h_z)
        n = jnp.tanh(i_n + r * h_n)
        h_new = (1.0 - z) * n + z * hs
        hs = jnp.where(tt < qlen, h_new, hs)
    out_ref[...] = hs


def _gru_final_hidden(emb, qlen, w_ih, w_hh, b_ih, b_hh):
    b, t, e = emb.shape
    hdim = w_hh.shape[1]
    gb = 1024 if b % 1024 == 0 else b
    return pl.pallas_call(
        _gru_kernel,
        out_shape=jax.ShapeDtypeStruct((b, hdim), jnp.float32),
        grid=(b // gb,),
        in_specs=[
            pl.BlockSpec((gb, t, e), lambda g: (g, 0, 0)),
            pl.BlockSpec((gb, 1), lambda g: (g, 0)),
            pl.BlockSpec((e, 3 * hdim), lambda g: (0, 0)),
            pl.BlockSpec((hdim, 3 * hdim), lambda g: (0, 0)),
            pl.BlockSpec((1, 3 * hdim), lambda g: (0, 0)),
            pl.BlockSpec((1, 3 * hdim), lambda g: (0, 0)),
        ],
        out_specs=pl.BlockSpec((gb, hdim), lambda g: (g, 0)),
        compiler_params=pltpu.CompilerParams(dimension_semantics=("parallel",)),
    )(emb, qlen.reshape(b, 1), jnp.transpose(w_ih), jnp.transpose(w_hh),
      b_ih.reshape(1, -1), b_hh.reshape(1, -1))


def kernel(wembed, gru_w_ih, gru_w_hh, gru_b_ih, gru_b_hh,
           adj_w1, adj_b1, adj_w2, adj_b2,
           gc1_conv_w, gc1_mean_rho, gc1_mean_theta, gc1_prec_rho, gc1_prec_theta,
           gc2_conv_w, gc2_mean_rho, gc2_mean_theta, gc2_prec_rho, gc2_prec_theta,
           out1_w, out1_b, out2_w, out2_b,
           question, image, qlen):
    b, k, _ = image.shape
    cb = 64 if b % 64 == 0 else 1

    emb = wembed[question]
    qenc = _gru_final_hidden(emb, qlen, gru_w_ih, gru_w_hh, gru_b_ih, gru_b_hh)

    adjacency = _adjacency(image, qenc, adj_w1, adj_b1, adj_w2, adj_b2, cb)

    bb = image[:, :, -4:]
    centre = bb[:, :, :2] + 0.5 * (bb[:, :, 2:] - bb[:, :, :2])
    cc = jnp.transpose(centre, (0, 2, 1))

    gc1 = {"conv_w": gc1_conv_w, "mean_rho": gc1_mean_rho,
           "mean_theta": gc1_mean_theta, "prec_rho": gc1_prec_rho,
           "prec_theta": gc1_prec_theta}
    gc2 = {"conv_w": gc2_conv_w, "mean_rho": gc2_mean_rho,
           "mean_theta": gc2_mean_theta, "prec_rho": gc2_prec_rho,
           "prec_theta": gc2_prec_theta}
    logits = _gc_head(adjacency, image, cc, qenc, gc1, gc2,
                      out1_w, out1_b, out2_w, out2_b, cb)
    return logits, adjacency


# speedup vs baseline: 1.1069x; 1.0163x over previous
import math

import jax
import jax.numpy as jnp
from jax import lax
from jax.experimental import pallas as pl
from jax.experimental.pallas import tpu as pltpu

_NB = 16
_M = 8


def _adj_kernel(x_ref, q_ref, w1a_ref, w1b_ref, b1_ref, w2_ref, b2_ref, out_ref):
    cb, k, d = x_ref.shape
    x = x_ref[...].astype(jnp.bfloat16).reshape(cb * k, d)
    h1 = jnp.dot(x, w1a_ref[...], preferred_element_type=jnp.float32)
    qp = jnp.dot(q_ref[...].astype(jnp.bfloat16), w1b_ref[...],
                 preferred_element_type=jnp.float32)
    h1 = h1.reshape(cb, k, h1.shape[-1]) + qp[:, None, :] + b1_ref[...]
    h1 = jnp.maximum(h1, 0.0).astype(jnp.bfloat16).reshape(cb * k, -1)
    h2 = jnp.dot(h1, w2_ref[...], preferred_element_type=jnp.float32) + b2_ref[...]
    h2 = jnp.maximum(h2, 0.0).astype(jnp.bfloat16)
    h2 = h2.reshape(cb, k, h2.shape[-1])
    for i in range(cb):
        hb = h2[i]
        out_ref[i] = lax.dot_general(hb, hb, (((1,), (1,)), ((), ())),
                                     preferred_element_type=jnp.float32)


def _adjacency(image, qenc, w1, b1, w2, b2, cb):
    b, k, d = image.shape
    h1 = w1.shape[1]
    w1a = w1[:d].astype(jnp.bfloat16)
    w1b = w1[d:].astype(jnp.bfloat16)
    return pl.pallas_call(
        _adj_kernel,
        out_shape=jax.ShapeDtypeStruct((b, k, k), jnp.float32),
        grid=(b // cb,),
        in_specs=[
            pl.BlockSpec((cb, k, d), lambda g: (g, 0, 0)),
            pl.BlockSpec((cb, qenc.shape[-1]), lambda g: (g, 0)),
            pl.BlockSpec(w1a.shape, lambda g: (0, 0)),
            pl.BlockSpec(w1b.shape, lambda g: (0, 0)),
            pl.BlockSpec((1, h1), lambda g: (0, 0)),
            pl.BlockSpec((h1, h1), lambda g: (0, 0)),
            pl.BlockSpec((1, h1), lambda g: (0, 0)),
        ],
        out_specs=pl.BlockSpec((cb, k, k), lambda g: (g, 0, 0)),
        compiler_params=pltpu.CompilerParams(dimension_semantics=("parallel",)),
    )(image, qenc, w1a, w1b, b1.reshape(1, h1), w2.astype(jnp.bfloat16),
      b2.reshape(1, h1))


def _tile_lanes(x, n):
    return jnp.concatenate([x] * n, axis=1)


def _topk_mask(adj_all, k):
    r = adj_all.shape[0]
    iota = lax.broadcasted_iota(jnp.int32, (r, k), 1)
    sel = jnp.zeros((r, k), jnp.float32)
    cur = adj_all
    for _ in range(_NB):
        jstar = jnp.argmax(cur, axis=1, keepdims=True)
        first = iota == jstar
        sel = jnp.where(first, 1.0, sel)
        cur = jnp.where(first, -jnp.inf, cur)
    return sel


def _gauss_block(rho_t, theta_t, g):
    dr = rho_t - g[0:1]
    first = jnp.abs(theta_t - g[1:2])
    ang = jnp.minimum(first, 2.0 * math.pi - first)
    return jnp.exp2(dr * dr * g[2:3] + ang * ang * g[3:4])


def _wsum(w_all, k):
    s = w_all[:, 0:k]
    for m in range(1, _M):
        s = s + w_all[:, m * k:(m + 1) * k]
    return s


def _fused_kernel(feats_ref, cc_ref, qenc_ref,
                  w1a_ref, w1b_ref, ab1_ref, aw2_ref, ab2_ref,
                  g1_ref, g2_ref, wc1_ref, wc2_ref, bm1_ref, bm2_ref,
                  w1_ref, b1_ref, w2_ref, b2_ref, out_ref, adj_out_ref):
    cb, k, d_in = feats_ref.shape

    x = feats_ref[...].astype(jnp.bfloat16).reshape(cb * k, d_in)
    ah1 = jnp.dot(x, w1a_ref[...], preferred_element_type=jnp.float32)
    qp = jnp.dot(qenc_ref[...].astype(jnp.bfloat16), w1b_ref[...],
                 preferred_element_type=jnp.float32)
    ah1 = ah1.reshape(cb, k, ah1.shape[-1]) + qp[:, None, :] + ab1_ref[...]
    ah1 = jnp.maximum(ah1, 0.0).astype(jnp.bfloat16).reshape(cb * k, -1)
    ah2 = jnp.dot(ah1, aw2_ref[...], preferred_element_type=jnp.float32) + ab2_ref[...]
    ah2 = jnp.maximum(ah2, 0.0).astype(jnp.bfloat16).reshape(cb, k, -1)
    adjs = []
    for i in range(cb):
        hb = ah2[i]
        adj_i = lax.dot_general(hb, hb, (((1,), (1,)), ((), ())),
                                preferred_element_type=jnp.float32)
        adj_out_ref[i] = adj_i
        adjs.append(adj_i)
    adj_all = jnp.concatenate(adjs, axis=0)

    sel_all = _topk_mask(adj_all, k)
    rowmax = jnp.max(adj_all, axis=1, keepdims=True)
    e = sel_all * jnp.exp(adj_all - rowmax)
    a1_all = e / jnp.sum(e, axis=1, keepdims=True)

    rhos, thetas = [], []
    for i in range(cb):
        cc = cc_ref[i]
        cxm = jnp.broadcast_to(cc[0:1, :], (k, k))
        cym = jnp.broadcast_to(cc[1:2, :], (k, k))
        dx = jnp.transpose(cxm) - cxm
        dy = jnp.transpose(cym) - cym
        rhos.append(jnp.sqrt(dx * dx + dy * dy))
        thetas.append(jnp.arctan2(dx, dy))
    rho_t = _tile_lanes(jnp.concatenate(rhos, axis=0), _M)
    theta_t = _tile_lanes(jnp.concatenate(thetas, axis=0), _M)

    w1_all = _gauss_block(rho_t, theta_t, g1_ref[...])
    w2_all = _gauss_block(rho_t, theta_t, g2_ref[...])
    b1_stack = w1_all * _tile_lanes(a1_all / (_wsum(w1_all, k) + 1e-20), _M)
    b2_stack = w2_all * _tile_lanes(sel_all / (_wsum(w2_all, k) + 1e-20), _M)

    proj1 = jnp.dot(feats_ref[...].reshape(cb * k, d_in), wc1_ref[...],
                    preferred_element_type=jnp.float32)
    bm1 = bm1_ref[...]
    bm2 = bm2_ref[...]
    hg1s = []
    for i in range(cb):
        p = jnp.concatenate([proj1[i * k:(i + 1) * k]] * _M, axis=0) * bm1
        o = jnp.dot(b1_stack[i * k:(i + 1) * k], p,
                    preferred_element_type=jnp.float32)
        hg1s.append(jnp.maximum(o, 0.0))
    hg1_all = jnp.concatenate(hg1s, axis=0)

    proj2 = jnp.dot(hg1_all, wc2_ref[...],
                    preferred_element_type=jnp.float32)
    gate_rows = []
    for i in range(cb):
        p = jnp.concatenate([proj2[i * k:(i + 1) * k]] * _M, axis=0) * bm2
        o = jnp.dot(b2_stack[i * k:(i + 1) * k], p,
                    preferred_element_type=jnp.float32)
        gate_rows.append(jnp.max(jnp.maximum(o, 0.0), axis=0, keepdims=True))

    q = jnp.maximum(qenc_ref[...], 0.0)
    h = q * jnp.concatenate(gate_rows, axis=0)
    h1 = jnp.dot(h, w1_ref[...], preferred_element_type=jnp.float32) + b1_ref[...]
    h1 = jnp.maximum(h1, 0.0)
    out_ref[...] = jnp.dot(h1, w2_ref[...],
                           preferred_element_type=jnp.float32) + b2_ref[...]


def _gauss_rows(mean_rho, mean_theta, prec_rho, prec_theta, k):
    c = -0.5 * math.log2(math.e)
    cr = c / (1e-14 + prec_rho ** 2)
    ct = c / (1e-14 + prec_theta ** 2)
    return jnp.stack([jnp.repeat(mean_rho, k), jnp.repeat(mean_theta, k),
                      jnp.repeat(cr, k), jnp.repeat(ct, k)], axis=0)


def _block_mask(k, dout):
    opk = dout // _M
    eye = jnp.eye(_M, dtype=jnp.float32)
    return jnp.repeat(jnp.repeat(eye, k, axis=0), opk, axis=1)


def _fused(image, cc, qenc, aw1, ab1, aw2, ab2, gc1, gc2, w1, b1, w2, b2, cb):
    b, k, d_in = image.shape
    h1a = aw1.shape[1]
    w1a = aw1[:d_in].astype(jnp.bfloat16)
    w1b = aw1[d_in:].astype(jnp.bfloat16)
    wc1 = jnp.transpose(gc1["conv_w"], (1, 0, 2)).reshape(d_in, -1)
    d1 = wc1.shape[1]
    wc2 = jnp.transpose(gc2["conv_w"], (1, 0, 2)).reshape(d1, -1)
    h = wc2.shape[1]
    o = w1.shape[0]
    g1 = _gauss_rows(gc1["mean_rho"], gc1["mean_theta"], gc1["prec_rho"],
                     gc1["prec_theta"], k)
    g2 = _gauss_rows(gc2["mean_rho"], gc2["mean_theta"], gc2["prec_rho"],
                     gc2["prec_theta"], k)
    bm1 = _block_mask(k, d1)
    bm2 = _block_mask(k, h)
    return pl.pallas_call(
        _fused_kernel,
        out_shape=(jax.ShapeDtypeStruct((b, o), jnp.float32),
                   jax.ShapeDtypeStruct((b, k, k), jnp.float32)),
        grid=(b // cb,),
        in_specs=[
            pl.BlockSpec((cb, k, d_in), lambda g: (g, 0, 0)),
            pl.BlockSpec((cb, 2, k), lambda g: (g, 0, 0)),
            pl.BlockSpec((cb, h), lambda g: (g, 0)),
            pl.BlockSpec(w1a.shape, lambda g: (0, 0)),
            pl.BlockSpec(w1b.shape, lambda g: (0, 0)),
            pl.BlockSpec((1, h1a), lambda g: (0, 0)),
            pl.BlockSpec((h1a, h1a), lambda g: (0, 0)),
            pl.BlockSpec((1, h1a), lambda g: (0, 0)),
            pl.BlockSpec((4, _M * k), lambda g: (0, 0)),
            pl.BlockSpec((4, _M * k), lambda g: (0, 0)),
            pl.BlockSpec((d_in, d1), lambda g: (0, 0)),
            pl.BlockSpec((d1, h), lambda g: (0, 0)),
            pl.BlockSpec((_M * k, d1), lambda g: (0, 0)),
            pl.BlockSpec((_M * k, h), lambda g: (0, 0)),
            pl.BlockSpec((h, o), lambda g: (0, 0)),
            pl.BlockSpec((1, o), lambda g: (0, 0)),
            pl.BlockSpec((o, o), lambda g: (0, 0)),
            pl.BlockSpec((1, o), lambda g: (0, 0)),
        ],
        out_specs=(pl.BlockSpec((cb, o), lambda g: (g, 0)),
                   pl.BlockSpec((cb, k, k), lambda g: (g, 0, 0))),
        compiler_params=pltpu.CompilerParams(dimension_semantics=("parallel",)),
    )(image, cc, qenc, w1a, w1b, ab1.reshape(1, h1a), aw2.astype(jnp.bfloat16),
      ab2.reshape(1, h1a), g1, g2, wc1, wc2, bm1, bm2,
      jnp.transpose(w1), b1.reshape(1, o), jnp.transpose(w2), b2.reshape(1, o))


def _gru_kernel(emb_ref, qlen_ref, wih_ref, whh_ref, bih_ref, bhh_ref, out_ref):
    gb, t, e = emb_ref.shape
    hdim = whh_ref.shape[0]
    qlen = qlen_ref[...]
    hs = jnp.zeros((gb, hdim), jnp.float32)
    for tt in range(t):
        xt = emb_ref[:, tt, :]
        gi = jnp.dot(xt, wih_ref[...], preferred_element_type=jnp.float32) + bih_ref[...]
        gh = jnp.dot(hs, whh_ref[...], preferred_element_type=jnp.float32) + bhh_ref[...]
        i_r, i_z, i_n = gi[:, :hdim], gi[:, hdim:2 * hdim], gi[:, 2 * hdim:]
        h_r, h_z, h_n = gh[:, :hdim], gh[:, hdim:2 * hdim], gh[:, 2 * hdim:]
        r = jax.nn.sigmoid(i_r + h_r)
        z = jax.nn.sigmoid(i_z + h_z)
        n = jnp.tanh(i_n + r * h_n)
        h_new = (1.0 - z) * n + z * hs
        hs = jnp.where(tt < qlen, h_new, hs)
    out_ref[...] = hs


def _gru_final_hidden(emb, qlen, w_ih, w_hh, b_ih, b_hh):
    b, t, e = emb.shape
    hdim = w_hh.shape[1]
    gb = 1024 if b % 1024 == 0 else b
    return pl.pallas_call(
        _gru_kernel,
        out_shape=jax.ShapeDtypeStruct((b, hdim), jnp.float32),
        grid=(b // gb,),
        in_specs=[
            pl.BlockSpec((gb, t, e), lambda g: (g, 0, 0)),
            pl.BlockSpec((gb, 1), lambda g: (g, 0)),
            pl.BlockSpec((e, 3 * hdim), lambda g: (0, 0)),
            pl.BlockSpec((hdim, 3 * hdim), lambda g: (0, 0)),
            pl.BlockSpec((1, 3 * hdim), lambda g: (0, 0)),
            pl.BlockSpec((1, 3 * hdim), lambda g: (0, 0)),
        ],
        out_specs=pl.BlockSpec((gb, hdim), lambda g: (g, 0)),
        compiler_params=pltpu.CompilerParams(dimension_semantics=("parallel",)),
    )(emb, qlen.reshape(b, 1), jnp.transpose(w_ih), jnp.transpose(w_hh),
      b_ih.reshape(1, -1), b_hh.reshape(1, -1))


def kernel(wembed, gru_w_ih, gru_w_hh, gru_b_ih, gru_b_hh,
           adj_w1, adj_b1, adj_w2, adj_b2,
           gc1_conv_w, gc1_mean_rho, gc1_mean_theta, gc1_prec_rho, gc1_prec_theta,
           gc2_conv_w, gc2_mean_rho, gc2_mean_theta, gc2_prec_rho, gc2_prec_theta,
           out1_w, out1_b, out2_w, out2_b,
           question, image, qlen):
    b, k, _ = image.shape
    cb = 32 if b % 32 == 0 else 1

    emb = wembed[question]
    qenc = _gru_final_hidden(emb, qlen, gru_w_ih, gru_w_hh, gru_b_ih, gru_b_hh)

    bb = image[:, :, -4:]
    centre = bb[:, :, :2] + 0.5 * (bb[:, :, 2:] - bb[:, :, :2])
    cc = jnp.transpose(centre, (0, 2, 1))

    gc1 = {"conv_w": gc1_conv_w, "mean_rho": gc1_mean_rho,
           "mean_theta": gc1_mean_theta, "prec_rho": gc1_prec_rho,
           "prec_theta": gc1_prec_theta}
    gc2 = {"conv_w": gc2_conv_w, "mean_rho": gc2_mean_rho,
           "mean_theta": gc2_mean_theta, "prec_rho": gc2_prec_rho,
           "prec_theta": gc2_prec_theta}
    logits, adjacency = _fused(image, cc, qenc, adj_w1, adj_b1, adj_w2, adj_b2,
                               gc1, gc2, out1_w, out1_b, out2_w, out2_b, cb)
    return logits, adjacency


# speedup vs baseline: 1.1160x; 1.0083x over previous
import math

import jax
import jax.numpy as jnp
from jax import lax
from jax.experimental import pallas as pl
from jax.experimental.pallas import tpu as pltpu

_NB = 16
_M = 8


def _adj_kernel(x_ref, q_ref, w1a_ref, w1b_ref, b1_ref, w2_ref, b2_ref, out_ref):
    cb, k, d = x_ref.shape
    x = x_ref[...].astype(jnp.bfloat16).reshape(cb * k, d)
    h1 = jnp.dot(x, w1a_ref[...], preferred_element_type=jnp.float32)
    qp = jnp.dot(q_ref[...].astype(jnp.bfloat16), w1b_ref[...],
                 preferred_element_type=jnp.float32)
    h1 = h1.reshape(cb, k, h1.shape[-1]) + qp[:, None, :] + b1_ref[...]
    h1 = jnp.maximum(h1, 0.0).astype(jnp.bfloat16).reshape(cb * k, -1)
    h2 = jnp.dot(h1, w2_ref[...], preferred_element_type=jnp.float32) + b2_ref[...]
    h2 = jnp.maximum(h2, 0.0).astype(jnp.bfloat16)
    h2 = h2.reshape(cb, k, h2.shape[-1])
    for i in range(cb):
        hb = h2[i]
        out_ref[i] = lax.dot_general(hb, hb, (((1,), (1,)), ((), ())),
                                     preferred_element_type=jnp.float32)


def _adjacency(image, qenc, w1, b1, w2, b2, cb):
    b, k, d = image.shape
    h1 = w1.shape[1]
    w1a = w1[:d].astype(jnp.bfloat16)
    w1b = w1[d:].astype(jnp.bfloat16)
    return pl.pallas_call(
        _adj_kernel,
        out_shape=jax.ShapeDtypeStruct((b, k, k), jnp.float32),
        grid=(b // cb,),
        in_specs=[
            pl.BlockSpec((cb, k, d), lambda g: (g, 0, 0)),
            pl.BlockSpec((cb, qenc.shape[-1]), lambda g: (g, 0)),
            pl.BlockSpec(w1a.shape, lambda g: (0, 0)),
            pl.BlockSpec(w1b.shape, lambda g: (0, 0)),
            pl.BlockSpec((1, h1), lambda g: (0, 0)),
            pl.BlockSpec((h1, h1), lambda g: (0, 0)),
            pl.BlockSpec((1, h1), lambda g: (0, 0)),
        ],
        out_specs=pl.BlockSpec((cb, k, k), lambda g: (g, 0, 0)),
        compiler_params=pltpu.CompilerParams(dimension_semantics=("parallel",)),
    )(image, qenc, w1a, w1b, b1.reshape(1, h1), w2.astype(jnp.bfloat16),
      b2.reshape(1, h1))


def _tile_lanes(x, n):
    return jnp.concatenate([x] * n, axis=1)


def _topk_mask(adj_all, k):
    r = adj_all.shape[0]
    iota = lax.broadcasted_iota(jnp.int32, (r, k), 1)
    sel = jnp.zeros((r, k), jnp.float32)
    cur = adj_all
    for _ in range(_NB):
        jstar = jnp.argmax(cur, axis=1, keepdims=True)
        first = iota == jstar
        sel = jnp.where(first, 1.0, sel)
        cur = jnp.where(first, -jnp.inf, cur)
    return sel


def _gauss_block(rho_t, theta_t, g):
    dr = rho_t - g[0:1]
    first = jnp.abs(theta_t - g[1:2])
    ang = jnp.minimum(first, 2.0 * math.pi - first)
    return jnp.exp2(dr * dr * g[2:3] + ang * ang * g[3:4])


def _wsum(w_all, k):
    s = w_all[:, 0:k]
    for m in range(1, _M):
        s = s + w_all[:, m * k:(m + 1) * k]
    return s


def _fused_kernel(feats_ref, cc_ref, qenc_ref,
                  w1a_ref, w1b_ref, ab1_ref, aw2_ref, ab2_ref,
                  g1_ref, g2_ref, wc1_ref, wc2_ref, bm1_ref, bm2_ref,
                  w1_ref, b1_ref, w2_ref, b2_ref, out_ref, adj_out_ref):
    cb, k, d_in = feats_ref.shape

    x = feats_ref[...].astype(jnp.bfloat16).reshape(cb * k, d_in)
    ah1 = jnp.dot(x, w1a_ref[...], preferred_element_type=jnp.float32)
    qp = jnp.dot(qenc_ref[...].astype(jnp.bfloat16), w1b_ref[...],
                 preferred_element_type=jnp.float32)
    ah1 = ah1.reshape(cb, k, ah1.shape[-1]) + qp[:, None, :] + ab1_ref[...]
    ah1 = jnp.maximum(ah1, 0.0).astype(jnp.bfloat16).reshape(cb * k, -1)
    ah2 = jnp.dot(ah1, aw2_ref[...], preferred_element_type=jnp.float32) + ab2_ref[...]
    ah2 = jnp.maximum(ah2, 0.0).astype(jnp.bfloat16).reshape(cb, k, -1)
    adjs = []
    for i in range(cb):
        hb = ah2[i]
        adj_i = lax.dot_general(hb, hb, (((1,), (1,)), ((), ())),
                                preferred_element_type=jnp.float32)
        adj_out_ref[i] = adj_i
        adjs.append(adj_i)
    adj_all = jnp.concatenate(adjs, axis=0)

    sel_all = _topk_mask(adj_all, k)
    rowmax = jnp.max(adj_all, axis=1, keepdims=True)
    e = sel_all * jnp.exp(adj_all - rowmax)
    a1_all = e / jnp.sum(e, axis=1, keepdims=True)

    rhos, thetas = [], []
    for i in range(cb):
        cc = cc_ref[i]
        cxm = jnp.broadcast_to(cc[0:1, :], (k, k))
        cym = jnp.broadcast_to(cc[1:2, :], (k, k))
        dx = jnp.transpose(cxm) - cxm
        dy = jnp.transpose(cym) - cym
        rhos.append(jnp.sqrt(dx * dx + dy * dy))
        thetas.append(jnp.arctan2(dx, dy))
    rho_t = _tile_lanes(jnp.concatenate(rhos, axis=0), _M)
    theta_t = _tile_lanes(jnp.concatenate(thetas, axis=0), _M)

    w1_all = _gauss_block(rho_t, theta_t, g1_ref[...])
    w2_all = _gauss_block(rho_t, theta_t, g2_ref[...])
    b1_stack = w1_all * _tile_lanes(a1_all / (_wsum(w1_all, k) + 1e-20), _M)
    b2_stack = w2_all * _tile_lanes(sel_all / (_wsum(w2_all, k) + 1e-20), _M)

    proj1 = jnp.dot(feats_ref[...].reshape(cb * k, d_in), wc1_ref[...],
                    preferred_element_type=jnp.float32)
    bm1 = bm1_ref[...]
    bm2 = bm2_ref[...]
    hg1s = []
    for i in range(cb):
        p = jnp.concatenate([proj1[i * k:(i + 1) * k]] * _M, axis=0) * bm1
        o = jnp.dot(b1_stack[i * k:(i + 1) * k], p,
                    preferred_element_type=jnp.float32)
        hg1s.append(jnp.maximum(o, 0.0))
    hg1_all = jnp.concatenate(hg1s, axis=0)

    proj2 = jnp.dot(hg1_all, wc2_ref[...],
                    preferred_element_type=jnp.float32)
    gate_rows = []
    for i in range(cb):
        p = jnp.concatenate([proj2[i * k:(i + 1) * k]] * _M, axis=0) * bm2
        o = jnp.dot(b2_stack[i * k:(i + 1) * k], p,
                    preferred_element_type=jnp.float32)
        gate_rows.append(jnp.max(jnp.maximum(o, 0.0), axis=0, keepdims=True))

    q = jnp.maximum(qenc_ref[...], 0.0)
    h = q * jnp.concatenate(gate_rows, axis=0)
    h1 = jnp.dot(h, w1_ref[...], preferred_element_type=jnp.float32) + b1_ref[...]
    h1 = jnp.maximum(h1, 0.0)
    out_ref[...] = jnp.dot(h1, w2_ref[...],
                           preferred_element_type=jnp.float32) + b2_ref[...]


def _gauss_rows(mean_rho, mean_theta, prec_rho, prec_theta, k):
    c = -0.5 * math.log2(math.e)
    cr = c / (1e-14 + prec_rho ** 2)
    ct = c / (1e-14 + prec_theta ** 2)
    return jnp.stack([jnp.repeat(mean_rho, k), jnp.repeat(mean_theta, k),
                      jnp.repeat(cr, k), jnp.repeat(ct, k)], axis=0)


def _block_mask(k, dout):
    opk = dout // _M
    eye = jnp.eye(_M, dtype=jnp.float32)
    return jnp.repeat(jnp.repeat(eye, k, axis=0), opk, axis=1)


def _fused(image, cc, qenc, aw1, ab1, aw2, ab2, gc1, gc2, w1, b1, w2, b2, cb):
    b, k, d_in = image.shape
    h1a = aw1.shape[1]
    w1a = aw1[:d_in].astype(jnp.bfloat16)
    w1b = aw1[d_in:].astype(jnp.bfloat16)
    wc1 = jnp.transpose(gc1["conv_w"], (1, 0, 2)).reshape(d_in, -1)
    d1 = wc1.shape[1]
    wc2 = jnp.transpose(gc2["conv_w"], (1, 0, 2)).reshape(d1, -1)
    h = wc2.shape[1]
    o = w1.shape[0]
    g1 = _gauss_rows(gc1["mean_rho"], gc1["mean_theta"], gc1["prec_rho"],
                     gc1["prec_theta"], k)
    g2 = _gauss_rows(gc2["mean_rho"], gc2["mean_theta"], gc2["prec_rho"],
                     gc2["prec_theta"], k)
    bm1 = _block_mask(k, d1)
    bm2 = _block_mask(k, h)
    return pl.pallas_call(
        _fused_kernel,
        out_shape=(jax.ShapeDtypeStruct((b, o), jnp.float32),
                   jax.ShapeDtypeStruct((b, k, k), jnp.float32)),
        grid=(b // cb,),
        in_specs=[
            pl.BlockSpec((cb, k, d_in), lambda g: (g, 0, 0)),
            pl.BlockSpec((cb, 2, k), lambda g: (g, 0, 0)),
            pl.BlockSpec((cb, h), lambda g: (g, 0)),
            pl.BlockSpec(w1a.shape, lambda g: (0, 0)),
            pl.BlockSpec(w1b.shape, lambda g: (0, 0)),
            pl.BlockSpec((1, h1a), lambda g: (0, 0)),
            pl.BlockSpec((h1a, h1a), lambda g: (0, 0)),
            pl.BlockSpec((1, h1a), lambda g: (0, 0)),
            pl.BlockSpec((4, _M * k), lambda g: (0, 0)),
            pl.BlockSpec((4, _M * k), lambda g: (0, 0)),
            pl.BlockSpec((d_in, d1), lambda g: (0, 0)),
            pl.BlockSpec((d1, h), lambda g: (0, 0)),
            pl.BlockSpec((_M * k, d1), lambda g: (0, 0)),
            pl.BlockSpec((_M * k, h), lambda g: (0, 0)),
            pl.BlockSpec((h, o), lambda g: (0, 0)),
            pl.BlockSpec((1, o), lambda g: (0, 0)),
            pl.BlockSpec((o, o), lambda g: (0, 0)),
            pl.BlockSpec((1, o), lambda g: (0, 0)),
        ],
        out_specs=(pl.BlockSpec((cb, o), lambda g: (g, 0)),
                   pl.BlockSpec((cb, k, k), lambda g: (g, 0, 0))),
        compiler_params=pltpu.CompilerParams(dimension_semantics=("parallel",)),
    )(image, cc, qenc, w1a, w1b, ab1.reshape(1, h1a), aw2.astype(jnp.bfloat16),
      ab2.reshape(1, h1a), g1, g2, wc1, wc2, bm1, bm2,
      jnp.transpose(w1), b1.reshape(1, o), jnp.transpose(w2), b2.reshape(1, o))


def _gru_kernel(emb_ref, qlen_ref, wih_ref, whh_ref, bih_ref, bhh_ref, out_ref):
    gb, t, e = emb_ref.shape
    hdim = whh_ref.shape[0]
    qlen = qlen_ref[...]
    hs = jnp.zeros((gb, hdim), jnp.float32)
    for tt in range(t):
        xt = emb_ref[:, tt, :]
        gi = jnp.dot(xt, wih_ref[...], preferred_element_type=jnp.float32) + bih_ref[...]
        gh = jnp.dot(hs, whh_ref[...], preferred_element_type=jnp.float32) + bhh_ref[...]
        i_r, i_z, i_n = gi[:, :hdim], gi[:, hdim:2 * hdim], gi[:, 2 * hdim:]
        h_r, h_z, h_n = gh[:, :hdim], gh[:, hdim:2 * hdim], gh[:, 2 * hdim:]
        r = jax.nn.sigmoid(i_r + h_r)
        z = jax.nn.sigmoid(i_z + h_z)
        n = jnp.tanh(i_n + r * h_n)
        h_new = (1.0 - z) * n + z * hs
        hs = jnp.where(tt < qlen, h_new, hs)
    out_ref[...] = hs


def _gru_final_hidden(emb, qlen, w_ih, w_hh, b_ih, b_hh):
    b, t, e = emb.shape
    hdim = w_hh.shape[1]
    gb = 1024 if b % 1024 == 0 else b
    return pl.pallas_call(
        _gru_kernel,
        out_shape=jax.ShapeDtypeStruct((b, hdim), jnp.float32),
        grid=(b // gb,),
        in_specs=[
            pl.BlockSpec((gb, t, e), lambda g: (g, 0, 0)),
            pl.BlockSpec((gb, 1), lambda g: (g, 0)),
            pl.BlockSpec((e, 3 * hdim), lambda g: (0, 0)),
            pl.BlockSpec((hdim, 3 * hdim), lambda g: (0, 0)),
            pl.BlockSpec((1, 3 * hdim), lambda g: (0, 0)),
            pl.BlockSpec((1, 3 * hdim), lambda g: (0, 0)),
        ],
        out_specs=pl.BlockSpec((gb, hdim), lambda g: (g, 0)),
        compiler_params=pltpu.CompilerParams(dimension_semantics=("parallel",)),
    )(emb, qlen.reshape(b, 1), jnp.transpose(w_ih), jnp.transpose(w_hh),
      b_ih.reshape(1, -1), b_hh.reshape(1, -1))


def kernel(wembed, gru_w_ih, gru_w_hh, gru_b_ih, gru_b_hh,
           adj_w1, adj_b1, adj_w2, adj_b2,
           gc1_conv_w, gc1_mean_rho, gc1_mean_theta, gc1_prec_rho, gc1_prec_theta,
           gc2_conv_w, gc2_mean_rho, gc2_mean_theta, gc2_prec_rho, gc2_prec_theta,
           out1_w, out1_b, out2_w, out2_b,
           question, image, qlen):
    b, k, _ = image.shape
    cb = 64 if b % 64 == 0 else 1

    emb = wembed[question]
    qenc = _gru_final_hidden(emb, qlen, gru_w_ih, gru_w_hh, gru_b_ih, gru_b_hh)

    bb = image[:, :, -4:]
    centre = bb[:, :, :2] + 0.5 * (bb[:, :, 2:] - bb[:, :, :2])
    cc = jnp.transpose(centre, (0, 2, 1))

    gc1 = {"conv_w": gc1_conv_w, "mean_rho": gc1_mean_rho,
           "mean_theta": gc1_mean_theta, "prec_rho": gc1_prec_rho,
           "prec_theta": gc1_prec_theta}
    gc2 = {"conv_w": gc2_conv_w, "mean_rho": gc2_mean_rho,
           "mean_theta": gc2_mean_theta, "prec_rho": gc2_prec_rho,
           "prec_theta": gc2_prec_theta}
    logits, adjacency = _fused(image, cc, qenc, adj_w1, adj_b1, adj_w2, adj_b2,
                               gc1, gc2, out1_w, out1_b, out2_w, out2_b, cb)
    return logits, adjacency


# speedup vs baseline: 1.2440x; 1.1147x over previous
import math

import jax
import jax.numpy as jnp
from jax import lax
from jax.experimental import pallas as pl
from jax.experimental.pallas import tpu as pltpu

_NB = 16
_M = 8


def _adj_kernel(x_ref, q_ref, w1a_ref, w1b_ref, b1_ref, w2_ref, b2_ref, out_ref):
    cb, k, d = x_ref.shape
    x = x_ref[...].astype(jnp.bfloat16).reshape(cb * k, d)
    h1 = jnp.dot(x, w1a_ref[...], preferred_element_type=jnp.float32)
    qp = jnp.dot(q_ref[...].astype(jnp.bfloat16), w1b_ref[...],
                 preferred_element_type=jnp.float32)
    h1 = h1.reshape(cb, k, h1.shape[-1]) + qp[:, None, :] + b1_ref[...]
    h1 = jnp.maximum(h1, 0.0).astype(jnp.bfloat16).reshape(cb * k, -1)
    h2 = jnp.dot(h1, w2_ref[...], preferred_element_type=jnp.float32) + b2_ref[...]
    h2 = jnp.maximum(h2, 0.0).astype(jnp.bfloat16)
    h2 = h2.reshape(cb, k, h2.shape[-1])
    for i in range(cb):
        hb = h2[i]
        out_ref[i] = lax.dot_general(hb, hb, (((1,), (1,)), ((), ())),
                                     preferred_element_type=jnp.float32)


def _adjacency(image, qenc, w1, b1, w2, b2, cb):
    b, k, d = image.shape
    h1 = w1.shape[1]
    w1a = w1[:d].astype(jnp.bfloat16)
    w1b = w1[d:].astype(jnp.bfloat16)
    return pl.pallas_call(
        _adj_kernel,
        out_shape=jax.ShapeDtypeStruct((b, k, k), jnp.float32),
        grid=(b // cb,),
        in_specs=[
            pl.BlockSpec((cb, k, d), lambda g: (g, 0, 0)),
            pl.BlockSpec((cb, qenc.shape[-1]), lambda g: (g, 0)),
            pl.BlockSpec(w1a.shape, lambda g: (0, 0)),
            pl.BlockSpec(w1b.shape, lambda g: (0, 0)),
            pl.BlockSpec((1, h1), lambda g: (0, 0)),
            pl.BlockSpec((h1, h1), lambda g: (0, 0)),
            pl.BlockSpec((1, h1), lambda g: (0, 0)),
        ],
        out_specs=pl.BlockSpec((cb, k, k), lambda g: (g, 0, 0)),
        compiler_params=pltpu.CompilerParams(dimension_semantics=("parallel",)),
    )(image, qenc, w1a, w1b, b1.reshape(1, h1), w2.astype(jnp.bfloat16),
      b2.reshape(1, h1))


def _tile_lanes(x, n):
    return jnp.concatenate([x] * n, axis=1)


def _topk_mask(adj_all, k):
    r = adj_all.shape[0]
    iota = lax.broadcasted_iota(jnp.int32, (r, k), 1)
    sel = jnp.zeros((r, k), jnp.float32)
    cur = adj_all
    for _ in range(_NB):
        jstar = jnp.argmax(cur, axis=1, keepdims=True)
        first = iota == jstar
        sel = jnp.where(first, 1.0, sel)
        cur = jnp.where(first, -jnp.inf, cur)
    return sel


def _gauss_block(rho_t, theta_t, g):
    dr = rho_t - g[0:1]
    first = jnp.abs(theta_t - g[1:2])
    ang = jnp.minimum(first, 2.0 * math.pi - first)
    return jnp.exp2(dr * dr * g[2:3] + ang * ang * g[3:4])


def _wsum(w_all, k):
    s = w_all[:, 0:k]
    for m in range(1, _M):
        s = s + w_all[:, m * k:(m + 1) * k]
    return s


def _fused_kernel(feats_ref, cc_ref, qenc_ref,
                  w1a_ref, w1b_ref, ab1_ref, aw2_ref, ab2_ref,
                  g1_ref, g2_ref, wc1_ref, wc2_ref, bm1_ref, bm2_ref,
                  w1_ref, b1_ref, w2_ref, b2_ref, out_ref, adj_out_ref):
    cb, k, d_in = feats_ref.shape

    x = feats_ref[...].astype(jnp.bfloat16).reshape(cb * k, d_in)
    ah1 = jnp.dot(x, w1a_ref[...], preferred_element_type=jnp.float32)
    qp = jnp.dot(qenc_ref[...].astype(jnp.bfloat16), w1b_ref[...],
                 preferred_element_type=jnp.float32)
    ah1 = ah1.reshape(cb, k, ah1.shape[-1]) + qp[:, None, :] + ab1_ref[...]
    ah1 = jnp.maximum(ah1, 0.0).astype(jnp.bfloat16).reshape(cb * k, -1)
    ah2 = jnp.dot(ah1, aw2_ref[...], preferred_element_type=jnp.float32) + ab2_ref[...]
    ah2 = jnp.maximum(ah2, 0.0).astype(jnp.bfloat16).reshape(cb, k, -1)
    adjs = []
    for i in range(cb):
        hb = ah2[i]
        adj_i = lax.dot_general(hb, hb, (((1,), (1,)), ((), ())),
                                preferred_element_type=jnp.float32)
        adj_out_ref[i] = adj_i
        adjs.append(adj_i)
    adj_all = jnp.concatenate(adjs, axis=0)

    sel_all = _topk_mask(adj_all, k)
    rowmax = jnp.max(adj_all, axis=1, keepdims=True)
    e = sel_all * jnp.exp(adj_all - rowmax)
    a1_all = e / jnp.sum(e, axis=1, keepdims=True)

    rhos, thetas = [], []
    for i in range(cb):
        cc = cc_ref[i]
        cxm = jnp.broadcast_to(cc[0:1, :], (k, k))
        cym = jnp.broadcast_to(cc[1:2, :], (k, k))
        dx = jnp.transpose(cxm) - cxm
        dy = jnp.transpose(cym) - cym
        rhos.append(jnp.sqrt(dx * dx + dy * dy))
        thetas.append(jnp.arctan2(dx, dy))
    rho_t = _tile_lanes(jnp.concatenate(rhos, axis=0), _M)
    theta_t = _tile_lanes(jnp.concatenate(thetas, axis=0), _M)

    w1_all = _gauss_block(rho_t, theta_t, g1_ref[...])
    w2_all = _gauss_block(rho_t, theta_t, g2_ref[...])
    b1_stack = w1_all * _tile_lanes(a1_all / (_wsum(w1_all, k) + 1e-20), _M)
    b2_stack = w2_all * _tile_lanes(sel_all / (_wsum(w2_all, k) + 1e-20), _M)

    proj1 = jnp.dot(feats_ref[...].reshape(cb * k, d_in), wc1_ref[...],
                    preferred_element_type=jnp.float32)
    bm1 = bm1_ref[...]
    bm2 = bm2_ref[...]
    hg1s = []
    for i in range(cb):
        p = jnp.concatenate([proj1[i * k:(i + 1) * k]] * _M, axis=0) * bm1
        o = jnp.dot(b1_stack[i * k:(i + 1) * k], p,
                    preferred_element_type=jnp.float32)
        hg1s.append(jnp.maximum(o, 0.0))
    hg1_all = jnp.concatenate(hg1s, axis=0)

    proj2 = jnp.dot(hg1_all, wc2_ref[...],
                    preferred_element_type=jnp.float32)
    gate_rows = []
    for i in range(cb):
        p = jnp.concatenate([proj2[i * k:(i + 1) * k]] * _M, axis=0) * bm2
        o = jnp.dot(b2_stack[i * k:(i + 1) * k], p,
                    preferred_element_type=jnp.float32)
        gate_rows.append(jnp.max(jnp.maximum(o, 0.0), axis=0, keepdims=True))

    q = jnp.maximum(qenc_ref[...], 0.0)
    h = q * jnp.concatenate(gate_rows, axis=0)
    h1 = jnp.dot(h, w1_ref[...], preferred_element_type=jnp.float32) + b1_ref[...]
    h1 = jnp.maximum(h1, 0.0)
    out_ref[...] = jnp.dot(h1, w2_ref[...],
                           preferred_element_type=jnp.float32) + b2_ref[...]


def _gauss_rows(mean_rho, mean_theta, prec_rho, prec_theta, k):
    c = -0.5 * math.log2(math.e)
    cr = c / (1e-14 + prec_rho ** 2)
    ct = c / (1e-14 + prec_theta ** 2)
    return jnp.stack([jnp.repeat(mean_rho, k), jnp.repeat(mean_theta, k),
                      jnp.repeat(cr, k), jnp.repeat(ct, k)], axis=0)


def _block_mask(k, dout):
    opk = dout // _M
    eye = jnp.eye(_M, dtype=jnp.float32)
    return jnp.repeat(jnp.repeat(eye, k, axis=0), opk, axis=1)


def _fused(image, cc, qenc, aw1, ab1, aw2, ab2, gc1, gc2, w1, b1, w2, b2, cb):
    b, k, d_in = image.shape
    h1a = aw1.shape[1]
    w1a = aw1[:d_in].astype(jnp.bfloat16)
    w1b = aw1[d_in:].astype(jnp.bfloat16)
    wc1 = jnp.transpose(gc1["conv_w"], (1, 0, 2)).reshape(d_in, -1)
    d1 = wc1.shape[1]
    wc2 = jnp.transpose(gc2["conv_w"], (1, 0, 2)).reshape(d1, -1)
    h = wc2.shape[1]
    o = w1.shape[0]
    g1 = _gauss_rows(gc1["mean_rho"], gc1["mean_theta"], gc1["prec_rho"],
                     gc1["prec_theta"], k)
    g2 = _gauss_rows(gc2["mean_rho"], gc2["mean_theta"], gc2["prec_rho"],
                     gc2["prec_theta"], k)
    bm1 = _block_mask(k, d1)
    bm2 = _block_mask(k, h)
    return pl.pallas_call(
        _fused_kernel,
        out_shape=(jax.ShapeDtypeStruct((b, o), jnp.float32),
                   jax.ShapeDtypeStruct((b, k, k), jnp.float32)),
        grid=(b // cb,),
        in_specs=[
            pl.BlockSpec((cb, k, d_in), lambda g: (g, 0, 0)),
            pl.BlockSpec((cb, 2, k), lambda g: (g, 0, 0)),
            pl.BlockSpec((cb, h), lambda g: (g, 0)),
            pl.BlockSpec(w1a.shape, lambda g: (0, 0)),
            pl.BlockSpec(w1b.shape, lambda g: (0, 0)),
            pl.BlockSpec((1, h1a), lambda g: (0, 0)),
            pl.BlockSpec((h1a, h1a), lambda g: (0, 0)),
            pl.BlockSpec((1, h1a), lambda g: (0, 0)),
            pl.BlockSpec((4, _M * k), lambda g: (0, 0)),
            pl.BlockSpec((4, _M * k), lambda g: (0, 0)),
            pl.BlockSpec((d_in, d1), lambda g: (0, 0)),
            pl.BlockSpec((d1, h), lambda g: (0, 0)),
            pl.BlockSpec((_M * k, d1), lambda g: (0, 0)),
            pl.BlockSpec((_M * k, h), lambda g: (0, 0)),
            pl.BlockSpec((h, o), lambda g: (0, 0)),
            pl.BlockSpec((1, o), lambda g: (0, 0)),
            pl.BlockSpec((o, o), lambda g: (0, 0)),
            pl.BlockSpec((1, o), lambda g: (0, 0)),
        ],
        out_specs=(pl.BlockSpec((cb, o), lambda g: (g, 0)),
                   pl.BlockSpec((cb, k, k), lambda g: (g, 0, 0))),
        compiler_params=pltpu.CompilerParams(dimension_semantics=("parallel",)),
    )(image, cc, qenc, w1a, w1b, ab1.reshape(1, h1a), aw2.astype(jnp.bfloat16),
      ab2.reshape(1, h1a), g1, g2, wc1, wc2, bm1, bm2,
      jnp.transpose(w1), b1.reshape(1, o), jnp.transpose(w2), b2.reshape(1, o))


def _gru_kernel(qt_ref, qlen_ref, wq_ref, whh_ref, bih_ref, bhh_ref, out_ref):
    t, gb = qt_ref.shape
    hdim = whh_ref.shape[1]
    vp = wq_ref.shape[1]
    io = lax.broadcasted_iota(jnp.int32, (vp, gb), 0)
    qlen_row = qlen_ref[...]
    bih = bih_ref[...]
    bhh = bhh_ref[...]
    ht = jnp.zeros((hdim, gb), jnp.float32)
    for tt in range(t):
        oh = jnp.where(io == qt_ref[tt:tt + 1, :], 1.0, 0.0)
        gi = jnp.dot(wq_ref[...], oh, preferred_element_type=jnp.float32) + bih
        gh = jnp.dot(whh_ref[...], ht, preferred_element_type=jnp.float32) + bhh
        r = jax.nn.sigmoid(gi[:hdim] + gh[:hdim])
        z = jax.nn.sigmoid(gi[hdim:2 * hdim] + gh[hdim:2 * hdim])
        n = jnp.tanh(gi[2 * hdim:] + r * gh[2 * hdim:])
        h_new = (1.0 - z) * n + z * ht
        ht = jnp.where(tt < qlen_row, h_new, ht)
    out_ref[...] = ht


def _gru_final_hidden(wembed, question, qlen, w_ih, w_hh, b_ih, b_hh):
    b, t = question.shape
    v = wembed.shape[0]
    hdim = w_hh.shape[1]
    vp = (v + 63) // 64 * 64
    wq = jnp.pad(w_ih @ jnp.transpose(wembed), ((0, 0), (0, vp - v)))
    gb = 1024 if b % 1024 == 0 else b
    qenc_t = pl.pallas_call(
        _gru_kernel,
        out_shape=jax.ShapeDtypeStruct((hdim, b), jnp.float32),
        grid=(b // gb,),
        in_specs=[
            pl.BlockSpec((t, gb), lambda g: (0, g)),
            pl.BlockSpec((1, gb), lambda g: (0, g)),
            pl.BlockSpec((3 * hdim, vp), lambda g: (0, 0)),
            pl.BlockSpec((3 * hdim, hdim), lambda g: (0, 0)),
            pl.BlockSpec((3 * hdim, 1), lambda g: (0, 0)),
            pl.BlockSpec((3 * hdim, 1), lambda g: (0, 0)),
        ],
        out_specs=pl.BlockSpec((hdim, gb), lambda g: (0, g)),
        compiler_params=pltpu.CompilerParams(dimension_semantics=("parallel",)),
    )(jnp.transpose(question), qlen.reshape(1, b), wq, w_hh,
      b_ih.reshape(-1, 1), b_hh.reshape(-1, 1))
    return jnp.transpose(qenc_t)


def kernel(wembed, gru_w_ih, gru_w_hh, gru_b_ih, gru_b_hh,
           adj_w1, adj_b1, adj_w2, adj_b2,
           gc1_conv_w, gc1_mean_rho, gc1_mean_theta, gc1_prec_rho, gc1_prec_theta,
           gc2_conv_w, gc2_mean_rho, gc2_mean_theta, gc2_prec_rho, gc2_prec_theta,
           out1_w, out1_b, out2_w, out2_b,
           question, image, qlen):
    b, k, _ = image.shape
    cb = 64 if b % 64 == 0 else 1

    qenc = _gru_final_hidden(wembed, question, qlen,
                             gru_w_ih, gru_w_hh, gru_b_ih, gru_b_hh)

    bb = image[:, :, -4:]
    centre = bb[:, :, :2] + 0.5 * (bb[:, :, 2:] - bb[:, :, :2])
    cc = jnp.transpose(centre, (0, 2, 1))

    gc1 = {"conv_w": gc1_conv_w, "mean_rho": gc1_mean_rho,
           "mean_theta": gc1_mean_theta, "prec_rho": gc1_prec_rho,
           "prec_theta": gc1_prec_theta}
    gc2 = {"conv_w": gc2_conv_w, "mean_rho": gc2_mean_rho,
           "mean_theta": gc2_mean_theta, "prec_rho": gc2_prec_rho,
           "prec_theta": gc2_prec_theta}
    logits, adjacency = _fused(image, cc, qenc, adj_w1, adj_b1, adj_w2, adj_b2,
                               gc1, gc2, out1_w, out1_b, out2_w, out2_b, cb)
    return logits, adjacency
